# baseline (speedup 1.0000x reference)
import jax
import jax.numpy as jnp
from jax import lax
from jax.experimental import pallas as pl
from jax.experimental.pallas import tpu as pltpu
from jax.experimental.pallas import tpu_sc as plsc

CONTEXT = 2048
EMB = 1024
BATCH = 8
NWORK = 32
TROWS = CONTEXT // NWORK
KC = 64
NCH = TROWS // KC


def _body(seq_hbm, table_hbm, out_hbm, seq_v, oidx, buf, gs0, gs1, ss0, ss1):
    gsems = (gs0, gs1)
    ssems = (ss0, ss1)
    cid = lax.axis_index("c")
    sid = lax.axis_index("s")
    w = sid * 2 + cid
    rbase = w * TROWS

    gd = []
    for c in range(NCH):
        rb = pl.multiple_of(rbase + KC * c, KC)
        gd.append(pltpu.async_copy(
            table_hbm.at[pl.ds(rb, KC)], buf.at[c], gsems[c]))

    pltpu.sync_copy(seq_hbm, seq_v)
    sv16 = seq_v[...]

    lane = lax.iota(jnp.int32, 16)
    for c in range(NCH):
        for b in range(BATCH):
            s_vec = lax.gather(
                sv16, jnp.full((16, 1), b, jnp.int32),
                lax.GatherDimensionNumbers(
                    offset_dims=(), collapsed_slice_dims=(0,),
                    start_index_map=(0,)),
                (1,), mode=lax.GatherScatterMode.PROMISE_IN_BOUNDS)
            for t in range(KC // 16):
                r = rbase + KC * c + 16 * t + lane
                oidx[c, b, pl.ds(16 * t, 16)] = (
                    b * CONTEXT + ((r - s_vec) & (CONTEXT - 1)))

    sd = []
    for c in range(NCH):
        gd[c].wait()
        for b in range(BATCH):
            sd.append(pltpu.async_copy(
                buf.at[c], out_hbm.at[oidx.at[c, b]], ssems[c]))
    for d in sd:
        d.wait()


_cache = {}


def _get_roll():
    if "k" not in _cache:
        mesh = plsc.VectorSubcoreMesh(core_axis_name="c", subcore_axis_name="s",
                                      num_cores=2, num_subcores=16)
        _cache["k"] = pl.kernel(
            _body,
            out_type=jax.ShapeDtypeStruct((BATCH * CONTEXT, EMB), jnp.float32),
            mesh=mesh,
            scratch_types=[
                pltpu.VMEM((16,), jnp.int32),
                pltpu.VMEM((NCH, BATCH, KC), jnp.int32),
                pltpu.VMEM((NCH, KC, EMB), jnp.float32),
                pltpu.SemaphoreType.DMA,
                pltpu.SemaphoreType.DMA,
                pltpu.SemaphoreType.DMA,
                pltpu.SemaphoreType.DMA,
            ],
        )
    return _cache["k"]


def kernel(seq_lengths, embeddings):
    seq16 = jnp.pad(seq_lengths.astype(jnp.int32), (0, 16 - BATCH))
    out = _get_roll()(seq16, embeddings)
    return out.reshape(BATCH, CONTEXT, EMB)
